# baseline (speedup 1.0000x reference)
import functools
import numpy as np

import jax
import jax.numpy as jnp
from jax import lax
from jax.experimental import pallas as pl
from jax.experimental.pallas import tpu as pltpu
from jax.experimental.pallas import tpu_sc as plsc

N_NODE = 50000
D_IN = 128
H = 64
NH = 2
DH = 32
E = 800000
N_LBL = 100000
ROW_BLK = 5000
INV_SQRT_DH = 1.0 / float(np.sqrt(DH))

NCORE = 2
NSUB = 16
CHUNK = 128
L_PAD = 102400
LCH_PER_W = L_PAD // CHUNK // (NCORE * NSUB)

_SC_MESH = plsc.VectorSubcoreMesh(core_axis_name="c", subcore_axis_name="s")
_SC_PARAMS = pltpu.CompilerParams(needs_layout_passes=False)


def _proj_body(x_ref, wl_ref, b_ref, w1_ref, w2_ref, w3_ref,
               h_ref, a_ref, b2_ref, c_ref):
    x = x_ref[...]
    h = jnp.maximum(
        jnp.dot(x, wl_ref[...], preferred_element_type=jnp.float32)
        + b_ref[...][None, :], 0.0)
    h_ref[...] = h
    a_ref[...] = jnp.dot(h, w1_ref[...], preferred_element_type=jnp.float32)
    b2_ref[...] = jnp.dot(h, w2_ref[...], preferred_element_type=jnp.float32)
    c_ref[...] = jnp.dot(h, w3_ref[...], preferred_element_type=jnp.float32)


def _proj(x, wl, b, w1, w2, w3):
    n = x.shape[0]
    grid = (n // ROW_BLK,)
    out_sd = jax.ShapeDtypeStruct((n, H), jnp.float32)
    row_spec = pl.BlockSpec((ROW_BLK, D_IN), lambda i: (i, 0))
    out_spec = pl.BlockSpec((ROW_BLK, H), lambda i: (i, 0))
    full = lambda s: pl.BlockSpec(s, lambda i: tuple(0 for _ in s))
    return pl.pallas_call(
        _proj_body,
        grid=grid,
        in_specs=[row_spec, full((D_IN, H)), full((H,)),
                  full((H, H)), full((H, H)), full((H, H))],
        out_specs=[out_spec, out_spec, out_spec, out_spec],
        out_shape=[out_sd, out_sd, out_sd, out_sd],
    )(x, wl, b, w1, w2, w3)


def _dec_sc_body(z_hbm, s_hbm, d_hbm, pred_hbm, sbuf, dbuf, zs, zd, outbuf):
    cc = lax.axis_index("c")
    sid = lax.axis_index("s")
    wid = cc * NSUB + sid
    lane = lax.iota(jnp.int32, 16)

    @pl.loop(0, LCH_PER_W)
    def _chunk(j):
        chunk = wid * LCH_PER_W + j
        pltpu.sync_copy(s_hbm.at[chunk], sbuf)
        pltpu.sync_copy(d_hbm.at[chunk], dbuf)
        pltpu.sync_copy(z_hbm.at[sbuf], zs)
        pltpu.sync_copy(z_hbm.at[dbuf], zd)

        @pl.loop(0, 8)
        def _grp(g):
            rows = g * 16 + lane
            acc = jnp.zeros((16,), jnp.float32)
            for d in range(H):
                scol = jnp.full((16,), d, jnp.int32)
                dcol = jnp.full((16,), H + d, jnp.int32)
                acc = acc + (plsc.load_gather(zs, [rows, scol]) *
                             plsc.load_gather(zd, [rows, dcol]))
            outbuf[pl.ds(g * 16, 16)] = 1.0 / (1.0 + jnp.exp(-acc))

        pltpu.sync_copy(outbuf, pred_hbm.at[pl.ds(chunk * CHUNK, CHUNK)])


_dec_sc = functools.partial(
    pl.kernel,
    _dec_sc_body,
    out_type=jax.ShapeDtypeStruct((L_PAD,), jnp.float32),
    mesh=_SC_MESH,
    compiler_params=_SC_PARAMS,
    scratch_types=[
        pltpu.VMEM((CHUNK,), jnp.int32),
        pltpu.VMEM((CHUNK,), jnp.int32),
        pltpu.VMEM((CHUNK, 2 * H), jnp.float32),
        pltpu.VMEM((CHUNK, 2 * H), jnp.float32),
        pltpu.VMEM((CHUNK,), jnp.float32),
    ],
)()


def _edge_phase(K, Q, V, ei, ew):
    src, dst = ei[0], ei[1]
    kk = K.reshape(-1, NH, DH)
    qq = Q.reshape(-1, NH, DH)
    vv = V.reshape(-1, NH, DH)
    alpha = (kk[src] * qq[dst]).sum(-1) * INV_SQRT_DH
    ex = jnp.exp(alpha)
    den = jax.ops.segment_sum(ex, dst, num_segments=N_NODE)
    msg = vv[src] * (ex * ew[:, None])[:, :, None]
    num = jax.ops.segment_sum(msg, dst, num_segments=N_NODE)
    out = num / (den[:, :, None] + 1e-9)
    return out.reshape(N_NODE, NH * DH)


def _pad_chunks(a, fill, n_pad):
    a = jnp.concatenate([a, jnp.full((n_pad,), fill, a.dtype)])
    return a.reshape(-1, CHUNK)


def kernel(x_question, x_answer, W_lin_q, b_lin_q, W_lin_a, b_lin_a,
           Wk_qa, Wq_qa, Wv_qa, Wk_aq, Wq_aq, Wv_aq, ew_qa, ew_aq,
           edge_index_qa, edge_index_aq, edge_label_index):
    h_q, K_qa, V_qa, Q_aq = _proj(x_question, W_lin_q, b_lin_q,
                                  Wk_qa, Wv_qa, Wq_aq)
    h_a, K_aq, V_aq, Q_qa = _proj(x_answer, W_lin_a, b_lin_a,
                                  Wk_aq, Wv_aq, Wq_qa)

    z_a = jax.nn.relu(h_a + _edge_phase(K_qa, Q_qa, V_qa,
                                        edge_index_qa, ew_qa))
    z_q = jax.nn.relu(h_q + _edge_phase(K_aq, Q_aq, V_aq,
                                        edge_index_aq, ew_aq))

    lpad = L_PAD - N_LBL
    s2 = _pad_chunks(edge_label_index[0], 0, lpad)
    d2 = _pad_chunks(edge_label_index[1], 0, lpad)
    z_all = jnp.concatenate([z_q, z_a], axis=1)
    pred = _dec_sc(z_all, s2, d2)
    return pred[:N_LBL]
